# baseline (speedup 1.0000x reference)
import functools

import jax
import jax.numpy as jnp
from jax import lax
from jax.experimental import pallas as pl
from jax.experimental.pallas import tpu as pltpu
from jax.experimental.pallas import tpu_sc as plsc

_F = 768
_H = 128
_ZD = 64
_QD = 32
_NZ = 1024
_NQ = 512
_TOK_BLK = 1024
_N_TOK = 8192
_HALF = _N_TOK // 2
_DP = 128


def _gelu(v):
    return jax.nn.gelu(v)


def _enc_body(x_ref, we1_ref, be1_ref, we2_ref, be2_ref, wz_ref, bz_ref,
              cbzt_ref, ze_ref, zidx_ref):
    x = x_ref[...]
    h = _gelu(jnp.dot(x, we1_ref[...], preferred_element_type=jnp.float32)
              + be1_ref[...])
    h = _gelu(jnp.dot(h, we2_ref[...], preferred_element_type=jnp.float32)
              + be2_ref[...])
    z_e = (jnp.dot(h, wz_ref[...], preferred_element_type=jnp.float32)
           + bz_ref[...])
    cbzt = cbzt_ref[...]
    csq = jnp.sum(cbzt * cbzt, axis=0, keepdims=True)
    zsq = jnp.sum(z_e * z_e, axis=1, keepdims=True)
    d2 = (zsq + csq) - 2.0 * jnp.dot(
        z_e, cbzt, preferred_element_type=jnp.float32)
    minv = jnp.min(d2, axis=1, keepdims=True)
    iota_z = lax.broadcasted_iota(jnp.int32, d2.shape, 1)
    idx_z = jnp.min(jnp.where(d2 == minv, iota_z, _NZ), axis=1,
                    keepdims=True)
    ze_ref[...] = z_e
    zidx_ref[...] = idx_z


def _mid_dec_body(x_ref, zqp_ref, ze_ref, wq_ref, bq_ref, cbq_ref, cbqt_ref,
                  wd1_ref, bd1_ref, wd2_ref, bd2_ref, wo_ref, bo_ref,
                  zq_ref, qq_ref, qidx_ref, acc_ref):
    i = pl.program_id(0)
    z_q = zqp_ref[...][:, :_ZD]
    q_e = (jnp.dot(z_q, wq_ref[...], preferred_element_type=jnp.float32)
           + bq_ref[...])
    cbqt = cbqt_ref[...]
    csq_q = jnp.sum(cbqt * cbqt, axis=0, keepdims=True)
    qsq = jnp.sum(q_e * q_e, axis=1, keepdims=True)
    d2q = (qsq + csq_q) - 2.0 * jnp.dot(
        q_e, cbqt, preferred_element_type=jnp.float32)
    minv_q = jnp.min(d2q, axis=1, keepdims=True)
    iota_q = lax.broadcasted_iota(jnp.int32, d2q.shape, 1)
    idx_q = jnp.min(jnp.where(d2q == minv_q, iota_q, _NQ), axis=1,
                    keepdims=True)
    oh_q = (iota_q == idx_q).astype(jnp.float32)
    q_q = jnp.dot(oh_q, cbq_ref[...], preferred_element_type=jnp.float32)

    bf = jnp.bfloat16
    r = _gelu(jnp.dot(q_q.astype(bf), wd1_ref[...].astype(bf),
                      preferred_element_type=jnp.float32) + bd1_ref[...])
    r = _gelu(jnp.dot(r.astype(bf), wd2_ref[...].astype(bf),
                      preferred_element_type=jnp.float32) + bd2_ref[...])
    x_rec = (jnp.dot(r.astype(bf), wo_ref[...].astype(bf),
                     preferred_element_type=jnp.float32) + bo_ref[...])

    dr = x_rec - x_ref[...]
    dz = z_q - ze_ref[...]
    dq = q_q - q_e
    rs = jnp.sum(dr * dr)
    zs = jnp.sum(dz * dz)
    qs = jnp.sum(dq * dq)

    zq_ref[...] = z_q
    qq_ref[...] = q_q
    qidx_ref[...] = idx_q
    lane = lax.broadcasted_iota(jnp.int32, (1, 128), 1)
    vec = (jnp.where(lane == 0, rs, 0.0)
           + jnp.where(lane == 1, zs, 0.0)
           + jnp.where(lane == 2, qs, 0.0))

    @pl.when(i == 0)
    def _init():
        acc_ref[...] = vec

    @pl.when(i > 0)
    def _accum():
        acc_ref[...] = acc_ref[...] + vec


def _make_sc_gather(n_rows, n_tok):
    info = plsc.get_sparse_core_info()
    nw = info.num_cores * info.num_subcores
    b_per_w = n_tok // nw
    n_chunk = max(1, b_per_w // 32)
    chunk = b_per_w // n_chunk
    mesh = plsc.VectorSubcoreMesh(core_axis_name="c", subcore_axis_name="s")

    @functools.partial(
        pl.kernel, mesh=mesh,
        out_type=jax.ShapeDtypeStruct((n_tok, _DP), jnp.float32),
        scratch_types=[
            pltpu.VMEM((b_per_w,), jnp.int32),
            pltpu.VMEM((b_per_w, _DP), jnp.float32),
            pltpu.VMEM_SHARED((n_rows, _DP), jnp.float32),
            pltpu.SemaphoreType.DMA,
        ],
    )
    def g(table_hbm, idx_hbm, out_hbm, idx_v, rows_v, tbl_sh, sem):
        sid = lax.axis_index("s")
        wid = sid * info.num_cores + lax.axis_index("c")
        base = wid * b_per_w

        @pl.when(sid == 0)
        def _stage():
            pltpu.sync_copy(table_hbm, tbl_sh)

        pltpu.sync_copy(idx_hbm.at[pl.ds(base, b_per_w)], idx_v)
        plsc.subcore_barrier()
        cps = [pltpu.async_copy(
                   tbl_sh.at[idx_v.at[pl.ds(c * chunk, chunk)]],
                   rows_v.at[pl.ds(c * chunk, chunk)], sem)
               for c in range(n_chunk)]
        for cp in cps:
            cp.wait()
        pltpu.sync_copy(rows_v, out_hbm.at[pl.ds(base, b_per_w)])

    return g


def kernel(x, W_e1, b_e1, W_e2, b_e2, W_z, b_z, cb_z, W_q, b_q, cb_q,
           W_d1, b_d1, W_d2, b_d2, W_o, b_o):
    B, S, F = x.shape
    N = B * S
    xf = x.reshape(N, F)
    T = _TOK_BLK
    half_blocks = _HALF // T
    grid = (half_blocks,)
    full = lambda shape: pl.BlockSpec(shape, lambda i: (0, 0))
    params = pltpu.CompilerParams(dimension_semantics=("arbitrary",))

    cbz_pad = jnp.concatenate(
        [cb_z, jnp.zeros((_NZ, _DP - _ZD), jnp.float32)], axis=1)

    enc_args = (W_e1.T, b_e1[None, :], W_e2.T, b_e2[None, :], W_z.T,
                b_z[None, :], cb_z.T)
    dec_args = (W_q.T, b_q[None, :], cb_q, cb_q.T,
                W_d1.T, b_d1[None, :], W_d2.T, b_d2[None, :],
                W_o.T, b_o[None, :])

    def run_a(h):
        off = h * half_blocks
        return pl.pallas_call(
            _enc_body,
            grid=grid,
            in_specs=[
                pl.BlockSpec((T, F), lambda i: (i + off, 0)),
                full((F, 64)), full((1, 64)),
                full((64, _H)), full((1, _H)),
                full((_H, _ZD)), full((1, _ZD)),
                full((_ZD, _NZ)),
            ],
            out_specs=(
                pl.BlockSpec((T, _ZD), lambda i: (i, 0)),
                pl.BlockSpec((T, 1), lambda i: (i, 0)),
            ),
            out_shape=(
                jax.ShapeDtypeStruct((_HALF, _ZD), jnp.float32),
                jax.ShapeDtypeStruct((_HALF, 1), jnp.int32),
            ),
            compiler_params=params,
        )(xf, *enc_args)

    def run_bd(h, z_q_pad, z_e):
        off = h * half_blocks
        return pl.pallas_call(
            _mid_dec_body,
            grid=grid,
            in_specs=[
                pl.BlockSpec((T, F), lambda i: (i + off, 0)),
                pl.BlockSpec((T, _DP), lambda i: (i, 0)),
                pl.BlockSpec((T, _ZD), lambda i: (i, 0)),
                full((_ZD, _QD)), full((1, _QD)),
                full((_NQ, _QD)), full((_QD, _NQ)),
                full((_QD, 64)), full((1, 64)),
                full((64, _H)), full((1, _H)),
                full((_H, _F)), full((1, _F)),
            ],
            out_specs=(
                pl.BlockSpec((T, _ZD), lambda i: (i, 0)),
                pl.BlockSpec((T, _QD), lambda i: (i, 0)),
                pl.BlockSpec((T, 1), lambda i: (i, 0)),
                pl.BlockSpec((1, 128), lambda i: (0, 0)),
            ),
            out_shape=(
                jax.ShapeDtypeStruct((_HALF, _ZD), jnp.float32),
                jax.ShapeDtypeStruct((_HALF, _QD), jnp.float32),
                jax.ShapeDtypeStruct((_HALF, 1), jnp.int32),
                jax.ShapeDtypeStruct((1, 128), jnp.float32),
            ),
            compiler_params=params,
        )(xf, z_q_pad, z_e, *dec_args)

    gather = _make_sc_gather(_NZ, _HALF)

    z_e0, z_idx0 = run_a(0)
    z_e1, z_idx1 = run_a(1)
    z_q_pad0 = gather(cbz_pad, z_idx0.reshape(_HALF))
    z_q_pad1 = gather(cbz_pad, z_idx1.reshape(_HALF))
    z_q0, q_q0, q_idx0, parts0 = run_bd(0, z_q_pad0, z_e0)
    z_q1, q_q1, q_idx1, parts1 = run_bd(1, z_q_pad1, z_e1)

    parts = parts0 + parts1
    z_q = jnp.concatenate([z_q0, z_q1], axis=0)
    q_q = jnp.concatenate([q_q0, q_q1], axis=0)
    z_idx = jnp.concatenate([z_idx0, z_idx1], axis=0)
    q_idx = jnp.concatenate([q_idx0, q_idx1], axis=0)

    loss = (parts[0, 0] / (N * _F)
            + 0.5 * (parts[0, 1] / (N * _ZD) + parts[0, 2] / (N * _QD)))
    return (z_q.reshape(B, S, _ZD), q_q.reshape(B, S, _QD),
            z_idx.reshape(B, S), q_idx.reshape(B, S), loss)


# speedup vs baseline: 1.1394x; 1.1394x over previous
import functools

import jax
import jax.numpy as jnp
from jax import lax
from jax.experimental import pallas as pl
from jax.experimental.pallas import tpu as pltpu
from jax.experimental.pallas import tpu_sc as plsc

_F = 768
_H = 128
_ZD = 64
_QD = 32
_NZ = 1024
_NQ = 512
_TOK_BLK = 2048
_N_TOK = 8192
_DP = 128


def _gelu(v):
    return jax.nn.gelu(v)


def _enc_body(x_ref, we1_ref, be1_ref, we2_ref, be2_ref, wz_ref, bz_ref,
              cbzt_ref, ze_ref, zidx_ref):
    x = x_ref[...]
    h = _gelu(jnp.dot(x, we1_ref[...], preferred_element_type=jnp.float32)
              + be1_ref[...])
    h = _gelu(jnp.dot(h, we2_ref[...], preferred_element_type=jnp.float32)
              + be2_ref[...])
    z_e = (jnp.dot(h, wz_ref[...], preferred_element_type=jnp.float32)
           + bz_ref[...])
    cbzt = cbzt_ref[...]
    csq = jnp.sum(cbzt * cbzt, axis=0, keepdims=True)
    zsq = jnp.sum(z_e * z_e, axis=1, keepdims=True)
    d2 = (zsq + csq) - 2.0 * jnp.dot(
        z_e, cbzt, preferred_element_type=jnp.float32)
    minv = jnp.min(d2, axis=1, keepdims=True)
    iota_z = lax.broadcasted_iota(jnp.int32, d2.shape, 1)
    idx_z = jnp.min(jnp.where(d2 == minv, iota_z, _NZ), axis=1,
                    keepdims=True)
    ze_ref[...] = z_e
    zidx_ref[...] = idx_z


def _mid_dec_body(x_ref, zqp_ref, ze_ref, wq_ref, bq_ref, cbq_ref, cbqt_ref,
                  wd1_ref, bd1_ref, wd2_ref, bd2_ref, wo_ref, bo_ref,
                  zq_ref, qq_ref, qidx_ref, acc_ref):
    i = pl.program_id(0)
    z_q = zqp_ref[...][:, :_ZD]
    q_e = (jnp.dot(z_q, wq_ref[...], preferred_element_type=jnp.float32)
           + bq_ref[...])
    cbqt = cbqt_ref[...]
    csq_q = jnp.sum(cbqt * cbqt, axis=0, keepdims=True)
    qsq = jnp.sum(q_e * q_e, axis=1, keepdims=True)
    d2q = (qsq + csq_q) - 2.0 * jnp.dot(
        q_e, cbqt, preferred_element_type=jnp.float32)
    minv_q = jnp.min(d2q, axis=1, keepdims=True)
    iota_q = lax.broadcasted_iota(jnp.int32, d2q.shape, 1)
    idx_q = jnp.min(jnp.where(d2q == minv_q, iota_q, _NQ), axis=1,
                    keepdims=True)
    oh_q = (iota_q == idx_q).astype(jnp.float32)
    q_q = jnp.dot(oh_q, cbq_ref[...], preferred_element_type=jnp.float32)

    bf = jnp.bfloat16
    r = _gelu(jnp.dot(q_q.astype(bf), wd1_ref[...].astype(bf),
                      preferred_element_type=jnp.float32) + bd1_ref[...])
    r = _gelu(jnp.dot(r.astype(bf), wd2_ref[...].astype(bf),
                      preferred_element_type=jnp.float32) + bd2_ref[...])
    x_rec = (jnp.dot(r.astype(bf), wo_ref[...].astype(bf),
                     preferred_element_type=jnp.float32) + bo_ref[...])

    dr = x_rec - x_ref[...]
    dz = z_q - ze_ref[...]
    dq = q_q - q_e
    rs = jnp.sum(dr * dr)
    zs = jnp.sum(dz * dz)
    qs = jnp.sum(dq * dq)

    zq_ref[...] = z_q
    qq_ref[...] = q_q
    qidx_ref[...] = idx_q
    lane = lax.broadcasted_iota(jnp.int32, (1, 128), 1)
    vec = (jnp.where(lane == 0, rs, 0.0)
           + jnp.where(lane == 1, zs, 0.0)
           + jnp.where(lane == 2, qs, 0.0))

    @pl.when(i == 0)
    def _init():
        acc_ref[...] = vec

    @pl.when(i > 0)
    def _accum():
        acc_ref[...] = acc_ref[...] + vec


def _make_sc_gather(n_rows):
    info = plsc.get_sparse_core_info()
    nw = info.num_cores * info.num_subcores
    b_per_w = _N_TOK // nw
    n_chunk = max(1, b_per_w // 32)
    chunk = b_per_w // n_chunk
    mesh = plsc.VectorSubcoreMesh(core_axis_name="c", subcore_axis_name="s")

    @functools.partial(
        pl.kernel, mesh=mesh,
        out_type=jax.ShapeDtypeStruct((_N_TOK, _DP), jnp.float32),
        scratch_types=[
            pltpu.VMEM((b_per_w,), jnp.int32),
            pltpu.VMEM((b_per_w, _DP), jnp.float32),
            pltpu.VMEM_SHARED((n_rows, _DP), jnp.float32),
            pltpu.SemaphoreType.DMA,
        ],
    )
    def g(table_hbm, idx_hbm, out_hbm, idx_v, rows_v, tbl_sh, sem):
        sid = lax.axis_index("s")
        wid = sid * info.num_cores + lax.axis_index("c")
        base = wid * b_per_w

        @pl.when(sid == 0)
        def _stage():
            pltpu.sync_copy(table_hbm, tbl_sh)

        pltpu.sync_copy(idx_hbm.at[pl.ds(base, b_per_w)], idx_v)
        plsc.subcore_barrier()
        cps = [pltpu.async_copy(
                   tbl_sh.at[idx_v.at[pl.ds(c * chunk, chunk)]],
                   rows_v.at[pl.ds(c * chunk, chunk)], sem)
               for c in range(n_chunk)]
        for cp in cps:
            cp.wait()
        pltpu.sync_copy(rows_v, out_hbm.at[pl.ds(base, b_per_w)])

    return g


def kernel(x, W_e1, b_e1, W_e2, b_e2, W_z, b_z, cb_z, W_q, b_q, cb_q,
           W_d1, b_d1, W_d2, b_d2, W_o, b_o):
    B, S, F = x.shape
    N = B * S
    xf = x.reshape(N, F)
    T = _TOK_BLK
    grid = (N // T,)
    full = lambda shape: pl.BlockSpec(shape, lambda i: (0, 0))
    params = pltpu.CompilerParams(dimension_semantics=("arbitrary",))

    cbz_pad = jnp.concatenate(
        [cb_z, jnp.zeros((_NZ, _DP - _ZD), jnp.float32)], axis=1)

    z_e, z_idx = pl.pallas_call(
        _enc_body,
        grid=grid,
        in_specs=[
            pl.BlockSpec((T, F), lambda i: (i, 0)),
            full((F, 64)), full((1, 64)),
            full((64, _H)), full((1, _H)),
            full((_H, _ZD)), full((1, _ZD)),
            full((_ZD, _NZ)),
        ],
        out_specs=(
            pl.BlockSpec((T, _ZD), lambda i: (i, 0)),
            pl.BlockSpec((T, 1), lambda i: (i, 0)),
        ),
        out_shape=(
            jax.ShapeDtypeStruct((N, _ZD), jnp.float32),
            jax.ShapeDtypeStruct((N, 1), jnp.int32),
        ),
        compiler_params=params,
    )(xf, W_e1.T, b_e1[None, :], W_e2.T, b_e2[None, :], W_z.T, b_z[None, :],
      cb_z.T)

    z_q_pad = _make_sc_gather(_NZ)(cbz_pad, z_idx.reshape(N))

    z_q, q_q, q_idx, parts = pl.pallas_call(
        _mid_dec_body,
        grid=grid,
        in_specs=[
            pl.BlockSpec((T, F), lambda i: (i, 0)),
            pl.BlockSpec((T, _DP), lambda i: (i, 0)),
            pl.BlockSpec((T, _ZD), lambda i: (i, 0)),
            full((_ZD, _QD)), full((1, _QD)),
            full((_NQ, _QD)), full((_QD, _NQ)),
            full((_QD, 64)), full((1, 64)),
            full((64, _H)), full((1, _H)),
            full((_H, _F)), full((1, _F)),
        ],
        out_specs=(
            pl.BlockSpec((T, _ZD), lambda i: (i, 0)),
            pl.BlockSpec((T, _QD), lambda i: (i, 0)),
            pl.BlockSpec((T, 1), lambda i: (i, 0)),
            pl.BlockSpec((1, 128), lambda i: (0, 0)),
        ),
        out_shape=(
            jax.ShapeDtypeStruct((N, _ZD), jnp.float32),
            jax.ShapeDtypeStruct((N, _QD), jnp.float32),
            jax.ShapeDtypeStruct((N, 1), jnp.int32),
            jax.ShapeDtypeStruct((1, 128), jnp.float32),
        ),
        compiler_params=params,
    )(xf, z_q_pad, z_e, W_q.T, b_q[None, :], cb_q, cb_q.T,
      W_d1.T, b_d1[None, :], W_d2.T, b_d2[None, :], W_o.T, b_o[None, :])

    loss = (parts[0, 0] / (N * _F)
            + 0.5 * (parts[0, 1] / (N * _ZD) + parts[0, 2] / (N * _QD)))
    return (z_q.reshape(B, S, _ZD), q_q.reshape(B, S, _QD),
            z_idx.reshape(B, S), q_idx.reshape(B, S), loss)


# speedup vs baseline: 1.3380x; 1.1744x over previous
import functools

import jax
import jax.numpy as jnp
from jax import lax
from jax.experimental import pallas as pl
from jax.experimental.pallas import tpu as pltpu

_F = 768
_H = 128
_ZD = 64
_QD = 32
_NZ = 1024
_NQ = 512
_TOK_BLK = 2048


def _gelu(v):
    return jax.nn.gelu(v)


def _fused_body(x_ref, we1_ref, be1_ref, we2_ref, be2_ref, wz_ref, bz_ref,
                cbz_ref, cbzt_ref, wq_ref, bq_ref, cbq_ref, cbqt_ref,
                wd1_ref, bd1_ref, wd2_ref, bd2_ref, wo_ref, bo_ref,
                zq_ref, qq_ref, zidx_ref, qidx_ref, acc_ref):
    i = pl.program_id(0)
    x = x_ref[...]

    h = _gelu(jnp.dot(x, we1_ref[...], preferred_element_type=jnp.float32)
              + be1_ref[...])
    h = _gelu(jnp.dot(h, we2_ref[...], preferred_element_type=jnp.float32)
              + be2_ref[...])
    z_e = (jnp.dot(h, wz_ref[...], preferred_element_type=jnp.float32)
           + bz_ref[...])

    cbzt = cbzt_ref[...]
    csq = jnp.sum(cbzt * cbzt, axis=0, keepdims=True)
    zsq = jnp.sum(z_e * z_e, axis=1, keepdims=True)
    d2 = (zsq + csq) - 2.0 * jnp.dot(
        z_e, cbzt, preferred_element_type=jnp.float32)
    minv = jnp.min(d2, axis=1, keepdims=True)
    iota_z = lax.broadcasted_iota(jnp.int32, d2.shape, 1)
    idx_z = jnp.min(jnp.where(d2 == minv, iota_z, _NZ), axis=1,
                    keepdims=True)
    oh_z = (iota_z == idx_z).astype(jnp.float32)
    z_q = jnp.dot(oh_z, cbz_ref[...],
                  preferred_element_type=jnp.float32)

    q_e = (jnp.dot(z_q, wq_ref[...], preferred_element_type=jnp.float32)
           + bq_ref[...])

    cbqt = cbqt_ref[...]
    csq_q = jnp.sum(cbqt * cbqt, axis=0, keepdims=True)
    qsq = jnp.sum(q_e * q_e, axis=1, keepdims=True)
    d2q = (qsq + csq_q) - 2.0 * jnp.dot(
        q_e, cbqt, preferred_element_type=jnp.float32)
    minv_q = jnp.min(d2q, axis=1, keepdims=True)
    iota_q = lax.broadcasted_iota(jnp.int32, d2q.shape, 1)
    idx_q = jnp.min(jnp.where(d2q == minv_q, iota_q, _NQ), axis=1,
                    keepdims=True)
    oh_q = (iota_q == idx_q).astype(jnp.float32)
    q_q = jnp.dot(oh_q, cbq_ref[...],
                  preferred_element_type=jnp.float32)

    bf = jnp.bfloat16
    r = _gelu(jnp.dot(q_q.astype(bf), wd1_ref[...].astype(bf),
                      preferred_element_type=jnp.float32)
              + bd1_ref[...])
    r = _gelu(jnp.dot(r.astype(bf), wd2_ref[...].astype(bf),
                      preferred_element_type=jnp.float32)
              + bd2_ref[...])
    x_rec = (jnp.dot(r.astype(bf), wo_ref[...].astype(bf),
                     preferred_element_type=jnp.float32)
             + bo_ref[...])

    dr = x_rec - x
    dz = z_q - z_e
    dq = q_q - q_e
    rs = jnp.sum(dr * dr)
    zs = jnp.sum(dz * dz)
    qs = jnp.sum(dq * dq)

    zq_ref[...] = z_q
    qq_ref[...] = q_q
    zidx_ref[...] = idx_z
    qidx_ref[...] = idx_q

    lane = lax.broadcasted_iota(jnp.int32, (1, 128), 1)
    vec = (jnp.where(lane == 0, rs, 0.0)
           + jnp.where(lane == 1, zs, 0.0)
           + jnp.where(lane == 2, qs, 0.0))

    @pl.when(i == 0)
    def _init():
        acc_ref[...] = vec

    @pl.when(i > 0)
    def _accum():
        acc_ref[...] = acc_ref[...] + vec


def kernel(x, W_e1, b_e1, W_e2, b_e2, W_z, b_z, cb_z, W_q, b_q, cb_q,
           W_d1, b_d1, W_d2, b_d2, W_o, b_o):
    B, S, F = x.shape
    N = B * S
    xf = x.reshape(N, F)
    T = _TOK_BLK
    grid = (N // T,)

    full = lambda shape: pl.BlockSpec(shape, lambda i: (0, 0))
    out_shapes = (
        jax.ShapeDtypeStruct((N, _ZD), jnp.float32),
        jax.ShapeDtypeStruct((N, _QD), jnp.float32),
        jax.ShapeDtypeStruct((N, 1), jnp.int32),
        jax.ShapeDtypeStruct((N, 1), jnp.int32),
        jax.ShapeDtypeStruct((1, 128), jnp.float32),
    )
    out_specs = (
        pl.BlockSpec((T, _ZD), lambda i: (i, 0)),
        pl.BlockSpec((T, _QD), lambda i: (i, 0)),
        pl.BlockSpec((T, 1), lambda i: (i, 0)),
        pl.BlockSpec((T, 1), lambda i: (i, 0)),
        pl.BlockSpec((1, 128), lambda i: (0, 0)),
    )
    in_specs = [
        pl.BlockSpec((T, F), lambda i: (i, 0)),
        full((F, 64)), full((1, 64)),
        full((64, _H)), full((1, _H)),
        full((_H, _ZD)), full((1, _ZD)),
        full((_NZ, _ZD)), full((_ZD, _NZ)),
        full((_ZD, _QD)), full((1, _QD)),
        full((_NQ, _QD)), full((_QD, _NQ)),
        full((_QD, 64)), full((1, 64)),
        full((64, _H)), full((1, _H)),
        full((_H, F)), full((1, F)),
    ]

    z_q, q_q, z_idx, q_idx, parts = pl.pallas_call(
        _fused_body,
        grid=grid,
        in_specs=in_specs,
        out_specs=out_specs,
        out_shape=out_shapes,
        compiler_params=pltpu.CompilerParams(
            dimension_semantics=("arbitrary",)),
    )(xf, W_e1.T, b_e1[None, :], W_e2.T, b_e2[None, :], W_z.T, b_z[None, :],
      cb_z, cb_z.T, W_q.T, b_q[None, :], cb_q, cb_q.T,
      W_d1.T, b_d1[None, :], W_d2.T, b_d2[None, :], W_o.T, b_o[None, :])

    loss = (parts[0, 0] / (N * F)
            + 0.5 * (parts[0, 1] / (N * _ZD) + parts[0, 2] / (N * _QD)))
    return (z_q.reshape(B, S, _ZD), q_q.reshape(B, S, _QD),
            z_idx.reshape(B, S), q_idx.reshape(B, S), loss)
